# baseline (speedup 1.0000x reference)
import jax
import jax.numpy as jnp
from jax.experimental import pallas as pl
from jax.experimental.pallas import tpu as pltpu

_BM = 200


def _agg_kernel(a0, a1, x_ref, wt_ref, b_ref, out0_ref, out1_ref):
    i = pl.program_id(0)
    half = x_ref.shape[0] // 2
    for a, out_ref, base in ((a0, out0_ref, 0), (a1, out1_ref, half)):
        s = jnp.dot(a[...], x_ref[...], preferred_element_type=jnp.float32)
        ego_rows = x_ref[pl.ds(base + i * _BM, _BM), :]
        y = ego_rows + s
        y = jnp.dot(y, wt_ref[...], preferred_element_type=jnp.float32)
        y = y + b_ref[...]
        out_ref[...] = jnp.where(y >= 0.0, y, 0.01 * y)


def kernel(ego_embeddings, A_in, W, b):
    N, D = ego_embeddings.shape
    nm = (N // 2) // _BM
    wt = W.T
    b2 = b.reshape(1, D)

    out0, out1 = pl.pallas_call(
        _agg_kernel,
        grid=(nm,),
        in_specs=[
            pl.BlockSpec((_BM, N), lambda i: (i, 0)),
            pl.BlockSpec((_BM, N), lambda i: (nm + i, 0)),
            pl.BlockSpec((N, D), lambda i: (0, 0)),
            pl.BlockSpec((D, D), lambda i: (0, 0)),
            pl.BlockSpec((1, D), lambda i: (0, 0)),
        ],
        out_specs=[
            pl.BlockSpec((_BM, D), lambda i: (i, 0)),
            pl.BlockSpec((_BM, D), lambda i: (i, 0)),
        ],
        out_shape=[
            jax.ShapeDtypeStruct((N // 2, D), jnp.float32),
            jax.ShapeDtypeStruct((N // 2, D), jnp.float32),
        ],
        compiler_params=pltpu.CompilerParams(
            dimension_semantics=("arbitrary",),
        ),
    )(A_in, A_in, ego_embeddings, wt, b2)
    return jnp.concatenate([out0, out1], axis=0)
